# baseline (speedup 1.0000x reference)
import functools

import jax
import jax.numpy as jnp
from jax import lax
from jax.experimental import pallas as pl
from jax.experimental.pallas import tpu as pltpu
from jax.experimental.pallas import tpu_sc as plsc

VOCAB = 30522
HIDDEN = 768
MAX_POS = 2048
BATCH = 4
SEQ = 2048
EPS = 1e-12

NC = 2
NS = 16
NW = NC * NS
POS_PER_W = SEQ // NW
CHUNK = 16
NBUF = 4
NQ = POS_PER_W // CHUNK
NVEC = HIDDEN // 16


def _rsqrt(x):
    i = lax.bitcast_convert_type(x, jnp.int32)
    i = jnp.int32(0x5F3759DF) - lax.shift_right_logical(i, 1)
    y = lax.bitcast_convert_type(i, jnp.float32)
    for _ in range(3):
        y = y * (1.5 - 0.5 * x * y * y)
    return y


def _body(ids_hbm, word_hbm, pos_hbm, gamma_hbm, beta_hbm, out_hbm,
          idx_v, rows, pos_v, xw_v, gamma_v, beta_v, st_v, qt_v, mr_v,
          gsems, wsems):
    wid = lax.axis_index("s") * NC + lax.axis_index("c")
    pos_base = wid * POS_PER_W

    pltpu.sync_copy(gamma_hbm, gamma_v)
    pltpu.sync_copy(beta_hbm, beta_v)
    for b in range(BATCH):
        pltpu.sync_copy(ids_hbm.at[pl.ds(b * SEQ + pos_base, POS_PER_W)],
                        idx_v.at[b])

    inv_h = jnp.float32(1.0 / HIDDEN)
    lanes = lax.iota(jnp.int32, 16)
    zero16 = jnp.zeros((16,), jnp.int32)
    one16 = jnp.ones((16,), jnp.int32)

    def out_base(b, h):
        return pl.multiple_of(b * SEQ + pos_base + h * CHUNK, CHUNK)

    def gather_copy(b, h):
        return pltpu.make_async_copy(
            word_hbm.at[idx_v.at[b, pl.ds(h * CHUNK, CHUNK)]],
            rows[b], gsems[b])

    def write_copy(b, h):
        return pltpu.make_async_copy(
            rows[b], out_hbm.at[pl.ds(out_base(b, h), CHUNK)], wsems[b])

    pltpu.sync_copy(pos_hbm.at[pl.ds(pos_base, CHUNK)], pos_v)
    gather_copy(0, 0).start()

    def outer(h, _):
        @pl.when(h >= 1)
        def _():
            pltpu.sync_copy(
                pos_hbm.at[pl.ds(pos_base + h * CHUNK, CHUNK)], pos_v)

        for i in range(BATCH):
            if i < BATCH - 1:
                @pl.when(h >= 1)
                def _(i=i):
                    write_copy(i + 1, h - 1).wait()
                gather_copy(i + 1, h).start()
            else:
                @pl.when(h < NQ - 1)
                def _(h=h):
                    write_copy(0, h).wait()
                    gather_copy(0, h + 1).start()

            gather_copy(i, h).wait()
            buf = rows[i]

            def abody(r, _, buf=buf):
                ss = [jnp.zeros((16,), jnp.float32) for _ in range(4)]
                qs = [jnp.zeros((16,), jnp.float32) for _ in range(4)]
                for j in range(NVEC):
                    x = (buf[r, pl.ds(16 * j, 16)]
                         + pos_v[r, pl.ds(16 * j, 16)])
                    xw_v[r, pl.ds(16 * j, 16)] = x
                    ss[j % 4] = ss[j % 4] + x
                    qs[j % 4] = qs[j % 4] + x * x
                s = (ss[0] + ss[1]) + (ss[2] + ss[3])
                q = (qs[0] + qs[1]) + (qs[2] + qs[3])
                col = jnp.full((16,), r, jnp.int32)
                plsc.store_scatter(st_v, [lanes, col], s)
                plsc.store_scatter(qt_v, [lanes, col], q)
                return 0

            lax.fori_loop(0, CHUNK, abody, 0)

            tot_s = st_v[0, :]
            tot_q = qt_v[0, :]
            for k in range(1, CHUNK):
                tot_s = tot_s + st_v[k, :]
                tot_q = tot_q + qt_v[k, :]
            mean = tot_s * inv_h
            var = tot_q * inv_h - mean * mean
            mr_v[0, :] = mean
            mr_v[1, :] = _rsqrt(var + EPS)

            def cbody(g, _, buf=buf):
                mvs = []
                rvs = []
                for k in range(8):
                    rcol = jnp.full((16,), g * 8 + k, jnp.int32)
                    mvs.append(plsc.load_gather(mr_v, [zero16, rcol]))
                    rvs.append(plsc.load_gather(mr_v, [one16, rcol]))

                def jbody(j, _):
                    js = pl.ds(pl.multiple_of(16 * j, 16), 16)
                    gj = gamma_v[js]
                    bj = beta_v[js]
                    for k in range(8):
                        row = g * 8 + k
                        t = (xw_v[row, js] - mvs[k]) * rvs[k]
                        buf[row, js] = t * gj + bj
                    return 0

                lax.fori_loop(0, NVEC, jbody, 0)
                return 0

            lax.fori_loop(0, 2, cbody, 0)

            write_copy(i, h).start()
        return 0

    lax.fori_loop(0, NQ, outer, 0)

    for i in range(BATCH):
        write_copy(i, NQ - 1).wait()


_mesh = plsc.VectorSubcoreMesh(core_axis_name="c", subcore_axis_name="s")

_kernel_call = functools.partial(
    pl.kernel,
    mesh=_mesh,
    compiler_params=pltpu.CompilerParams(needs_layout_passes=False),
    out_type=jax.ShapeDtypeStruct((BATCH * SEQ, HIDDEN), jnp.float32),
    scratch_types=[
        pltpu.VMEM((BATCH, POS_PER_W), jnp.int32),
        [pltpu.VMEM((CHUNK, HIDDEN), jnp.float32)] * NBUF,
        pltpu.VMEM((CHUNK, HIDDEN), jnp.float32),
        pltpu.VMEM((CHUNK, HIDDEN), jnp.float32),
        pltpu.VMEM((HIDDEN,), jnp.float32),
        pltpu.VMEM((HIDDEN,), jnp.float32),
        pltpu.VMEM((CHUNK, CHUNK), jnp.float32),
        pltpu.VMEM((CHUNK, CHUNK), jnp.float32),
        pltpu.VMEM((2, CHUNK), jnp.float32),
        [pltpu.SemaphoreType.DMA] * NBUF,
        [pltpu.SemaphoreType.DMA] * NBUF,
    ],
)(_body)


@jax.jit
def kernel(input_ids, word_emb, pos_emb, ln_gamma, ln_beta):
    ids_flat = jnp.reshape(input_ids.astype(jnp.int32), (BATCH * SEQ,))
    out = _kernel_call(ids_flat, word_emb, pos_emb, ln_gamma, ln_beta)
    return jnp.reshape(out, (BATCH, SEQ, HIDDEN))


# speedup vs baseline: 1.2566x; 1.2566x over previous
import functools

import jax
import jax.numpy as jnp
from jax import lax
from jax.experimental import pallas as pl
from jax.experimental.pallas import tpu as pltpu
from jax.experimental.pallas import tpu_sc as plsc

VOCAB = 30522
HIDDEN = 768
MAX_POS = 2048
BATCH = 4
SEQ = 2048
EPS = 1e-12

NC = 2
NS = 16
NW = NC * NS
POS_PER_W = SEQ // NW
CHUNK = 16
NQ = POS_PER_W // CHUNK
NVEC = HIDDEN // 16


def _rsqrt(x):
    i = lax.bitcast_convert_type(x, jnp.int32)
    i = jnp.int32(0x5F3759DF) - lax.shift_right_logical(i, 1)
    y = lax.bitcast_convert_type(i, jnp.float32)
    for _ in range(3):
        y = y * (1.5 - 0.5 * x * y * y)
    return y


def _body(ids_hbm, word_hbm, pos_hbm, gamma_hbm, beta_hbm, out_hbm,
          idx_v, rows, pos_v, gamma_v, beta_v, st_v, qt_v, mr_v,
          gsems, wsems):
    wid = lax.axis_index("s") * NC + lax.axis_index("c")
    pos_base = wid * POS_PER_W

    pltpu.sync_copy(gamma_hbm, gamma_v)
    pltpu.sync_copy(beta_hbm, beta_v)
    for b in range(BATCH):
        pltpu.sync_copy(ids_hbm.at[pl.ds(b * SEQ + pos_base, POS_PER_W)],
                        idx_v.at[b])

    inv_h = jnp.float32(1.0 / HIDDEN)
    lanes = lax.iota(jnp.int32, 16)
    zero16 = jnp.zeros((16,), jnp.int32)
    one16 = jnp.ones((16,), jnp.int32)

    def out_base(b, q):
        return pl.multiple_of(b * SEQ + pos_base + q * CHUNK, CHUNK)

    def gather_copy(b, q, grp):
        slot = grp * BATCH + b
        return pltpu.make_async_copy(
            word_hbm.at[idx_v.at[b, pl.ds(q * CHUNK, CHUNK)]],
            rows[slot], gsems[slot])

    def write_copy(b, q, grp):
        slot = grp * BATCH + b
        return pltpu.make_async_copy(
            rows[slot], out_hbm.at[pl.ds(out_base(b, q), CHUNK)],
            wsems[slot])

    pltpu.sync_copy(pos_hbm.at[pl.ds(pos_base, CHUNK)], pos_v)
    for b in range(BATCH):
        gather_copy(b, 0, 0).start()

    def quarter(q, grp):
        bufs = [rows[grp * BATCH + b] for b in range(BATCH)]

        @pl.when(q >= 1)
        def _():
            pltpu.sync_copy(
                pos_hbm.at[pl.ds(pos_base + q * CHUNK, CHUNK)], pos_v)

        for b in range(BATCH):
            gather_copy(b, q, grp).wait()

        def abody(r, _):
            ss = [jnp.zeros((16,), jnp.float32) for _ in range(BATCH)]
            qs = [jnp.zeros((16,), jnp.float32) for _ in range(BATCH)]
            for j in range(NVEC):
                pv = pos_v[r, pl.ds(16 * j, 16)]
                for b in range(BATCH):
                    x = bufs[b][r, pl.ds(16 * j, 16)] + pv
                    bufs[b][r, pl.ds(16 * j, 16)] = x
                    ss[b] = ss[b] + x
                    qs[b] = qs[b] + x * x
            for b in range(BATCH):
                col = jnp.full((16,), b * CHUNK + r, jnp.int32)
                plsc.store_scatter(st_v, [lanes, col], ss[b])
                plsc.store_scatter(qt_v, [lanes, col], qs[b])
            return 0

        lax.fori_loop(0, CHUNK, abody, 0)

        other = 1 - grp

        @pl.when(q >= 1)
        def _():
            for b in range(BATCH):
                write_copy(b, q - 1, other).wait()

        @pl.when(q < NQ - 1)
        def _():
            for b in range(BATCH):
                gather_copy(b, q + 1, other).start()

        for b in range(BATCH):
            gs = pl.ds(b * CHUNK, CHUNK)
            tot_s = st_v[0, gs]
            tot_q = qt_v[0, gs]
            for k in range(1, CHUNK):
                tot_s = tot_s + st_v[k, gs]
                tot_q = tot_q + qt_v[k, gs]
            mean = tot_s * inv_h
            var = tot_q * inv_h - mean * mean
            mr_v[0, gs] = mean
            mr_v[1, gs] = _rsqrt(var + EPS)

        for b in range(BATCH):
            def cbody(g, _, b=b):
                mvs = []
                rvs = []
                for k in range(8):
                    rcol = jnp.full((16,), b * CHUNK + g * 8 + k, jnp.int32)
                    mvs.append(plsc.load_gather(mr_v, [zero16, rcol]))
                    rvs.append(plsc.load_gather(mr_v, [one16, rcol]))

                def jbody(j, _):
                    js = pl.ds(pl.multiple_of(16 * j, 16), 16)
                    gj = gamma_v[js]
                    bj = beta_v[js]
                    for k in range(8):
                        row = g * 8 + k
                        t = (bufs[b][row, js] - mvs[k]) * rvs[k]
                        bufs[b][row, js] = t * gj + bj
                    return 0

                lax.fori_loop(0, NVEC, jbody, 0)
                return 0

            lax.fori_loop(0, 2, cbody, 0)

        for b in range(BATCH):
            write_copy(b, q, grp).start()

    def outer(qq, _):
        quarter(2 * qq, 0)
        quarter(2 * qq + 1, 1)
        return 0

    lax.fori_loop(0, NQ // 2, outer, 0)

    for b in range(BATCH):
        write_copy(b, NQ - 1, 1).wait()


_mesh = plsc.VectorSubcoreMesh(core_axis_name="c", subcore_axis_name="s")

_kernel_call = functools.partial(
    pl.kernel,
    mesh=_mesh,
    compiler_params=pltpu.CompilerParams(needs_layout_passes=False),
    out_type=jax.ShapeDtypeStruct((BATCH * SEQ, HIDDEN), jnp.float32),
    scratch_types=[
        pltpu.VMEM((BATCH, POS_PER_W), jnp.int32),
        [pltpu.VMEM((CHUNK, HIDDEN), jnp.float32)] * (2 * BATCH),
        pltpu.VMEM((CHUNK, HIDDEN), jnp.float32),
        pltpu.VMEM((HIDDEN,), jnp.float32),
        pltpu.VMEM((HIDDEN,), jnp.float32),
        pltpu.VMEM((CHUNK, BATCH * CHUNK), jnp.float32),
        pltpu.VMEM((CHUNK, BATCH * CHUNK), jnp.float32),
        pltpu.VMEM((2, BATCH * CHUNK), jnp.float32),
        [pltpu.SemaphoreType.DMA] * (2 * BATCH),
        [pltpu.SemaphoreType.DMA] * (2 * BATCH),
    ],
)(_body)


@jax.jit
def kernel(input_ids, word_emb, pos_emb, ln_gamma, ln_beta):
    ids_flat = jnp.reshape(input_ids.astype(jnp.int32), (BATCH * SEQ,))
    out = _kernel_call(ids_flat, word_emb, pos_emb, ln_gamma, ln_beta)
    return jnp.reshape(out, (BATCH, SEQ, HIDDEN))


# speedup vs baseline: 1.4248x; 1.1338x over previous
import functools

import jax
import jax.numpy as jnp
from jax import lax
from jax.experimental import pallas as pl
from jax.experimental.pallas import tpu as pltpu
from jax.experimental.pallas import tpu_sc as plsc

VOCAB = 30522
HIDDEN = 768
MAX_POS = 2048
BATCH = 4
SEQ = 2048
EPS = 1e-12

NC = 2
NS = 16
NW = NC * NS
POS_PER_W = SEQ // NW
CHUNK = 32
NBUF = 4
NQ = POS_PER_W // CHUNK
NVEC = HIDDEN // 16


def _rsqrt(x):
    i = lax.bitcast_convert_type(x, jnp.int32)
    i = jnp.int32(0x5F3759DF) - lax.shift_right_logical(i, 1)
    y = lax.bitcast_convert_type(i, jnp.float32)
    for _ in range(3):
        y = y * (1.5 - 0.5 * x * y * y)
    return y


def _body(ids_hbm, word_hbm, pos_hbm, gamma_hbm, beta_hbm, out_hbm,
          idx_v, rows, pos_v, gamma_v, beta_v, st_v, qt_v, mr_v,
          gsems, wsems):
    wid = lax.axis_index("s") * NC + lax.axis_index("c")
    pos_base = wid * POS_PER_W

    pltpu.sync_copy(gamma_hbm, gamma_v)
    pltpu.sync_copy(beta_hbm, beta_v)
    for b in range(BATCH):
        pltpu.sync_copy(ids_hbm.at[pl.ds(b * SEQ + pos_base, POS_PER_W)],
                        idx_v.at[b])

    inv_h = jnp.float32(1.0 / HIDDEN)
    lanes = lax.iota(jnp.int32, 16)
    zero16 = jnp.zeros((16,), jnp.int32)
    one16 = jnp.ones((16,), jnp.int32)

    def out_base(b, h):
        return pl.multiple_of(b * SEQ + pos_base + h * CHUNK, CHUNK)

    def gather_copy(b, h):
        return pltpu.make_async_copy(
            word_hbm.at[idx_v.at[b, pl.ds(h * CHUNK, CHUNK)]],
            rows[b], gsems[b])

    def write_copy(b, h):
        return pltpu.make_async_copy(
            rows[b], out_hbm.at[pl.ds(out_base(b, h), CHUNK)], wsems[b])

    pltpu.sync_copy(pos_hbm.at[pl.ds(pos_base, CHUNK)], pos_v)
    gather_copy(0, 0).start()

    def outer(h, _):
        @pl.when(h >= 1)
        def _():
            pltpu.sync_copy(
                pos_hbm.at[pl.ds(pos_base + h * CHUNK, CHUNK)], pos_v)

        for i in range(BATCH):
            if i < BATCH - 1:
                @pl.when(h >= 1)
                def _(i=i):
                    write_copy(i + 1, h - 1).wait()
                gather_copy(i + 1, h).start()
            else:
                @pl.when(h < NQ - 1)
                def _(h=h):
                    write_copy(0, h).wait()
                    gather_copy(0, h + 1).start()

            gather_copy(i, h).wait()
            buf = rows[i]

            def abody(r, _, buf=buf):
                ss = [jnp.zeros((16,), jnp.float32) for _ in range(4)]
                qs = [jnp.zeros((16,), jnp.float32) for _ in range(4)]
                for j in range(NVEC):
                    x = (buf[r, pl.ds(16 * j, 16)]
                         + pos_v[r, pl.ds(16 * j, 16)])
                    buf[r, pl.ds(16 * j, 16)] = x
                    ss[j % 4] = ss[j % 4] + x
                    qs[j % 4] = qs[j % 4] + x * x
                s = (ss[0] + ss[1]) + (ss[2] + ss[3])
                q = (qs[0] + qs[1]) + (qs[2] + qs[3])
                col = jnp.full((16,), r, jnp.int32)
                plsc.store_scatter(st_v, [lanes, col], s)
                plsc.store_scatter(qt_v, [lanes, col], q)
                return 0

            lax.fori_loop(0, CHUNK, abody, 0)

            for gg in range(CHUNK // 16):
                gsl = pl.ds(gg * 16, 16)
                tot_s = st_v[0, gsl]
                tot_q = qt_v[0, gsl]
                for k in range(1, 16):
                    tot_s = tot_s + st_v[k, gsl]
                    tot_q = tot_q + qt_v[k, gsl]
                mean = tot_s * inv_h
                var = tot_q * inv_h - mean * mean
                mr_v[0, gsl] = mean
                mr_v[1, gsl] = _rsqrt(var + EPS)

            def cbody(g, _, buf=buf):
                mvs = []
                rvs = []
                for k in range(8):
                    rcol = jnp.full((16,), g * 8 + k, jnp.int32)
                    mvs.append(plsc.load_gather(mr_v, [zero16, rcol]))
                    rvs.append(plsc.load_gather(mr_v, [one16, rcol]))

                def jbody(j, _):
                    js = pl.ds(pl.multiple_of(16 * j, 16), 16)
                    gj = gamma_v[js]
                    bj = beta_v[js]
                    for k in range(8):
                        row = g * 8 + k
                        t = (buf[row, js] - mvs[k]) * rvs[k]
                        buf[row, js] = t * gj + bj
                    return 0

                lax.fori_loop(0, NVEC, jbody, 0)
                return 0

            lax.fori_loop(0, CHUNK // 8, cbody, 0)

            write_copy(i, h).start()
        return 0

    lax.fori_loop(0, NQ, outer, 0)

    for i in range(BATCH):
        write_copy(i, NQ - 1).wait()


_mesh = plsc.VectorSubcoreMesh(core_axis_name="c", subcore_axis_name="s")

_kernel_call = functools.partial(
    pl.kernel,
    mesh=_mesh,
    compiler_params=pltpu.CompilerParams(needs_layout_passes=False),
    out_type=jax.ShapeDtypeStruct((BATCH * SEQ, HIDDEN), jnp.float32),
    scratch_types=[
        pltpu.VMEM((BATCH, POS_PER_W), jnp.int32),
        [pltpu.VMEM((CHUNK, HIDDEN), jnp.float32)] * NBUF,
        pltpu.VMEM((CHUNK, HIDDEN), jnp.float32),
        pltpu.VMEM((HIDDEN,), jnp.float32),
        pltpu.VMEM((HIDDEN,), jnp.float32),
        pltpu.VMEM((16, CHUNK), jnp.float32),
        pltpu.VMEM((16, CHUNK), jnp.float32),
        pltpu.VMEM((2, CHUNK), jnp.float32),
        [pltpu.SemaphoreType.DMA] * NBUF,
        [pltpu.SemaphoreType.DMA] * NBUF,
    ],
)(_body)


@jax.jit
def kernel(input_ids, word_emb, pos_emb, ln_gamma, ln_beta):
    ids_flat = jnp.reshape(input_ids.astype(jnp.int32), (BATCH * SEQ,))
    out = _kernel_call(ids_flat, word_emb, pos_emb, ln_gamma, ln_beta)
    return jnp.reshape(out, (BATCH, SEQ, HIDDEN))


# speedup vs baseline: 1.6502x; 1.1582x over previous
import functools

import jax
import jax.numpy as jnp
from jax import lax
from jax.experimental import pallas as pl
from jax.experimental.pallas import tpu as pltpu
from jax.experimental.pallas import tpu_sc as plsc

VOCAB = 30522
HIDDEN = 768
MAX_POS = 2048
BATCH = 4
SEQ = 2048
EPS = 1e-12

NC = 2
NS = 16
NW = NC * NS
POS_PER_W = SEQ // NW
CHUNK = 16
NBUF = 4
NQ = POS_PER_W // CHUNK
NVEC = HIDDEN // 16


def _rsqrt(x):
    i = lax.bitcast_convert_type(x, jnp.int32)
    i = jnp.int32(0x5F3759DF) - lax.shift_right_logical(i, 1)
    y = lax.bitcast_convert_type(i, jnp.float32)
    for _ in range(3):
        y = y * (1.5 - 0.5 * x * y * y)
    return y


def _body(ids_hbm, word_hbm, pos_hbm, gamma_hbm, beta_hbm, out_hbm,
          idx_v, rows, pos_v, st_v, qt_v, mr_v,
          gsems, wsems):
    wid = lax.axis_index("s") * NC + lax.axis_index("c")
    pos_base = wid * POS_PER_W

    for b in range(BATCH):
        pltpu.sync_copy(ids_hbm.at[pl.ds(b * SEQ + pos_base, POS_PER_W)],
                        idx_v.at[b])

    inv_h = jnp.float32(1.0 / HIDDEN)
    lanes = lax.iota(jnp.int32, 16)
    zero16 = jnp.zeros((16,), jnp.int32)
    one16 = jnp.ones((16,), jnp.int32)

    def out_base(b, h):
        return pl.multiple_of(b * SEQ + pos_base + h * CHUNK, CHUNK)

    def gather_copy(b, h):
        return pltpu.make_async_copy(
            word_hbm.at[idx_v.at[b, pl.ds(h * CHUNK, CHUNK)]],
            rows[b], gsems[b])

    def write_copy(b, h):
        return pltpu.make_async_copy(
            rows[b], out_hbm.at[pl.ds(out_base(b, h), CHUNK)], wsems[b])

    pltpu.sync_copy(pos_hbm.at[pl.ds(pos_base, CHUNK)], pos_v)
    gather_copy(0, 0).start()

    def outer(h, _):
        @pl.when(h >= 1)
        def _():
            pltpu.sync_copy(
                pos_hbm.at[pl.ds(pos_base + h * CHUNK, CHUNK)], pos_v)

        for i in range(BATCH):
            if i < BATCH - 1:
                @pl.when(h >= 1)
                def _(i=i):
                    write_copy(i + 1, h - 1).wait()
                gather_copy(i + 1, h).start()
            else:
                @pl.when(h < NQ - 1)
                def _(h=h):
                    write_copy(0, h).wait()
                    gather_copy(0, h + 1).start()

            gather_copy(i, h).wait()
            buf = rows[i]

            def abody(r, _, buf=buf):
                ss = [jnp.zeros((16,), jnp.float32) for _ in range(4)]
                qs = [jnp.zeros((16,), jnp.float32) for _ in range(4)]
                for j in range(NVEC):
                    x = (buf[r, pl.ds(16 * j, 16)]
                         + pos_v[r, pl.ds(16 * j, 16)])
                    buf[r, pl.ds(16 * j, 16)] = x
                    ss[j % 4] = ss[j % 4] + x
                    qs[j % 4] = qs[j % 4] + x * x
                s = (ss[0] + ss[1]) + (ss[2] + ss[3])
                q = (qs[0] + qs[1]) + (qs[2] + qs[3])
                col = jnp.full((16,), r, jnp.int32)
                plsc.store_scatter(st_v, [lanes, col], s)
                plsc.store_scatter(qt_v, [lanes, col], q)
                return 0

            lax.fori_loop(0, CHUNK, abody, 0)

            tot_s = st_v[0, :]
            tot_q = qt_v[0, :]
            for k in range(1, CHUNK):
                tot_s = tot_s + st_v[k, :]
                tot_q = tot_q + qt_v[k, :]
            mean = tot_s * inv_h
            var = tot_q * inv_h - mean * mean
            mr_v[0, :] = mean
            mr_v[1, :] = _rsqrt(var + EPS)

            def cbody(g, _, buf=buf):
                mvs = []
                rvs = []
                for k in range(8):
                    rcol = jnp.full((16,), g * 8 + k, jnp.int32)
                    mvs.append(plsc.load_gather(mr_v, [zero16, rcol]))
                    rvs.append(plsc.load_gather(mr_v, [one16, rcol]))

                def jbody(j, _):
                    js = pl.ds(pl.multiple_of(16 * j, 16), 16)
                    for k in range(8):
                        row = g * 8 + k
                        buf[row, js] = (buf[row, js] - mvs[k]) * rvs[k]
                    return 0

                lax.fori_loop(0, NVEC, jbody, 0)
                return 0

            lax.fori_loop(0, 2, cbody, 0)

            write_copy(i, h).start()
        return 0

    lax.fori_loop(0, NQ, outer, 0)

    for i in range(BATCH):
        write_copy(i, NQ - 1).wait()


_mesh = plsc.VectorSubcoreMesh(core_axis_name="c", subcore_axis_name="s")

_kernel_call = functools.partial(
    pl.kernel,
    mesh=_mesh,
    compiler_params=pltpu.CompilerParams(needs_layout_passes=False),
    out_type=jax.ShapeDtypeStruct((BATCH * SEQ, HIDDEN), jnp.float32),
    scratch_types=[
        pltpu.VMEM((BATCH, POS_PER_W), jnp.int32),
        [pltpu.VMEM((CHUNK, HIDDEN), jnp.float32)] * NBUF,
        pltpu.VMEM((CHUNK, HIDDEN), jnp.float32),
        pltpu.VMEM((CHUNK, CHUNK), jnp.float32),
        pltpu.VMEM((CHUNK, CHUNK), jnp.float32),
        pltpu.VMEM((2, CHUNK), jnp.float32),
        [pltpu.SemaphoreType.DMA] * NBUF,
        [pltpu.SemaphoreType.DMA] * NBUF,
    ],
)(_body)


@jax.jit
def kernel(input_ids, word_emb, pos_emb, ln_gamma, ln_beta):
    ids_flat = jnp.reshape(input_ids.astype(jnp.int32), (BATCH * SEQ,))
    out = _kernel_call(ids_flat, word_emb, pos_emb, ln_gamma, ln_beta)
    return jnp.reshape(out, (BATCH, SEQ, HIDDEN))
